# baseline (device time: 10846 ns/iter reference)
import jax
import jax.numpy as jnp
from jax import lax
from jax.experimental import pallas as pl
from jax.experimental.pallas import tpu as pltpu

N_DEV = 4


def kernel(x, W1, W2):
    m, k1 = x.shape
    _, h = W1.shape
    _, n = W2.shape

    def body(x_ref, w1_ref, w2_ref, out_ref, send_buf, comm_ref,
             send_sems, recv_sems):
        my = lax.axis_index("i")

        barrier = pltpu.get_barrier_semaphore()
        for k in range(1, N_DEV):
            peer = lax.rem(my + k, N_DEV)
            pl.semaphore_signal(
                barrier, inc=1,
                device_id=(peer,), device_id_type=pl.DeviceIdType.MESH,
            )
        pl.semaphore_wait(barrier, N_DEV - 1)

        xb = x_ref[...].astype(jnp.bfloat16)
        w1 = w1_ref[...].astype(jnp.bfloat16)
        hidden = lax.dot(xb, w1, preferred_element_type=jnp.float32)
        hidden = jnp.maximum(hidden, 0.0).astype(jnp.bfloat16)
        w2 = w2_ref[...].astype(jnp.bfloat16)
        partial = lax.dot(hidden, w2, preferred_element_type=jnp.float32)
        send_buf[...] = partial.astype(jnp.bfloat16)

        rdmas = []
        for k in range(1, N_DEV):
            dst = lax.rem(my + k, N_DEV)
            j = N_DEV - 1 - k
            rdma = pltpu.make_async_remote_copy(
                src_ref=send_buf,
                dst_ref=comm_ref.at[j],
                send_sem=send_sems.at[k - 1],
                recv_sem=recv_sems.at[j],
                device_id=(dst,),
                device_id_type=pl.DeviceIdType.MESH,
            )
            rdma.start()
            rdmas.append(rdma)

        for rdma in rdmas:
            rdma.wait_recv()
        acc = partial
        for j in range(N_DEV - 1):
            acc = acc + comm_ref[j].astype(jnp.float32)
        out_ref[...] = acc

        for rdma in rdmas:
            rdma.wait_send()

    return pl.pallas_call(
        body,
        out_shape=jax.ShapeDtypeStruct((m, n), jnp.float32),
        in_specs=[
            pl.BlockSpec(memory_space=pltpu.VMEM),
            pl.BlockSpec(memory_space=pltpu.VMEM),
            pl.BlockSpec(memory_space=pltpu.VMEM),
        ],
        out_specs=pl.BlockSpec(memory_space=pltpu.VMEM),
        scratch_shapes=[
            pltpu.VMEM((m, n), jnp.bfloat16),
            pltpu.VMEM((N_DEV - 1, m, n), jnp.bfloat16),
            pltpu.SemaphoreType.DMA((N_DEV - 1,)),
            pltpu.SemaphoreType.DMA((N_DEV - 1,)),
        ],
        compiler_params=pltpu.CompilerParams(collective_id=0),
    )(x, W1, W2)
